# Initial kernel scaffold; baseline (speedup 1.0000x reference)
import jax
import jax.numpy as jnp
from jax.experimental import pallas as pl


def kernel(inputs, edge_index, W_self0, W_neigh0, b0, W_self1, W_neigh1, b1):
    raise NotImplementedError("write your pallas kernel here")


# speedup vs baseline: 2.4227x; 2.4227x over previous
import jax
import jax.numpy as jnp
from jax import lax
from jax.experimental import pallas as pl
from jax.experimental.pallas import tpu as pltpu
from jax.experimental.pallas import tpu_sc as plsc

N = 10000
D = 256
E = 160000

LHALF = 128
NPAD = 10240
EPAD = 163840
KCH = 128
NSUB = 16
EDGES_PER_TILE = EPAD // NSUB
CHUNKS = EDGES_PER_TILE // KCH
STRIPE = NPAD // NSUB


def _make_sc_aggregate(compute_deg: bool):
    out_type = [jax.ShapeDtypeStruct((NPAD, LHALF), jnp.float32),
                jax.ShapeDtypeStruct((NPAD, LHALF), jnp.float32)]
    if compute_deg:
        out_type.append(jax.ShapeDtypeStruct((NPAD,), jnp.float32))

    scratch = [
        pltpu.VMEM((KCH,), jnp.int32),
        pltpu.VMEM((KCH,), jnp.int32),
        pltpu.VMEM((KCH, LHALF), jnp.float32),
        pltpu.VMEM_SHARED((NPAD, LHALF), jnp.float32),
        pltpu.SemaphoreType.DMA,
    ]
    if compute_deg:
        scratch += [
            pltpu.VMEM((NPAD,), jnp.float32),
            pltpu.VMEM((NSUB, STRIPE), jnp.float32),
            pltpu.VMEM((STRIPE,), jnp.float32),
            pltpu.VMEM_SHARED((NSUB * NPAD,), jnp.float32),
        ]

    def body(x0_hbm, x1_hbm, src_hbm, dst_hbm, zrow_hbm, zdeg_hbm, *rest):
        if compute_deg:
            agg0_out, agg1_out, deg_out = rest[0], rest[1], rest[2]
            (idx_v, dst_v, rows_v, agg_sh, sem,
             deg_local, red_buf, deg_red, deg_parts) = rest[3:]
        else:
            agg0_out, agg1_out = rest[0], rest[1]
            idx_v, dst_v, rows_v, agg_sh, sem = rest[2:]

        c = lax.axis_index("c")
        s = lax.axis_index("s")
        ebase = s * EDGES_PER_TILE
        stripe = pl.ds(s * STRIPE, STRIPE)

        pltpu.sync_copy(zrow_hbm, agg_sh.at[stripe])
        if compute_deg:
            @pl.when(c == 0)
            def _():
                pltpu.sync_copy(zdeg_hbm, deg_local)
        plsc.subcore_barrier()

        ones16 = jnp.full((16,), 1.0, dtype=jnp.float32)

        def chunk(i, carry):
            off = ebase + i * KCH
            pltpu.sync_copy(src_hbm.at[pl.ds(off, KCH)], idx_v)
            pltpu.sync_copy(dst_hbm.at[pl.ds(off, KCH)], dst_v)

            @pl.when(c == 0)
            def _():
                pltpu.async_copy(x0_hbm.at[idx_v], rows_v, sem).wait()

            @pl.when(c == 1)
            def _():
                pltpu.async_copy(x1_hbm.at[idx_v], rows_v, sem).wait()

            pltpu.sync_copy(rows_v, agg_sh.at[dst_v], add=True)
            if compute_deg:
                @pl.when(c == 0)
                def _():
                    for j in range(KCH // 16):
                        idx16 = dst_v[pl.ds(j * 16, 16)]
                        plsc.addupdate_scatter(deg_local, [idx16], ones16)
            return carry

        lax.fori_loop(0, CHUNKS, chunk, 0)

        if compute_deg:
            @pl.when(c == 0)
            def _():
                pltpu.sync_copy(deg_local,
                                deg_parts.at[pl.ds(s * NPAD, NPAD)])
        plsc.subcore_barrier()

        @pl.when(c == 0)
        def _():
            pltpu.sync_copy(agg_sh.at[stripe], agg0_out.at[stripe])

        @pl.when(c == 1)
        def _():
            pltpu.sync_copy(agg_sh.at[stripe], agg1_out.at[stripe])

        if compute_deg:
            @pl.when(c == 0)
            def _():
                for t in range(NSUB):
                    pltpu.sync_copy(
                        deg_parts.at[pl.ds(t * NPAD + s * STRIPE, STRIPE)],
                        red_buf.at[t])

                def red(j, carry):
                    sl = pl.ds(j * 16, 16)
                    acc = red_buf[0, sl]
                    for t in range(1, NSUB):
                        acc = acc + red_buf[t, sl]
                    deg_red[sl] = acc
                    return carry

                lax.fori_loop(0, STRIPE // 16, red, 0)
                pltpu.sync_copy(deg_red, deg_out.at[stripe])

    return pl.kernel(
        body,
        mesh=plsc.VectorSubcoreMesh(core_axis_name="c", subcore_axis_name="s"),
        out_type=out_type,
        scratch_types=scratch,
        compiler_params=pltpu.CompilerParams(needs_layout_passes=False),
    )


def _tc_layer_body(h_ref, agg_ref, deg_ref, wst_ref, wnt_ref, b_ref, out_ref):
    deg = jnp.maximum(deg_ref[...], 1.0)
    hn = agg_ref[...] / deg
    acc = jnp.dot(h_ref[...], wst_ref[...], preferred_element_type=jnp.float32)
    acc = acc + jnp.dot(hn, wnt_ref[...], preferred_element_type=jnp.float32)
    out_ref[...] = jnp.maximum(acc + b_ref[...], 0.0)


def _tc_layer(h, agg, deg, wst, wnt, b):
    BR = 1000
    grid = (N // BR,)
    return pl.pallas_call(
        _tc_layer_body,
        grid=grid,
        in_specs=[
            pl.BlockSpec((BR, D), lambda i: (i, 0)),
            pl.BlockSpec((BR, D), lambda i: (i, 0)),
            pl.BlockSpec((BR, 1), lambda i: (i, 0)),
            pl.BlockSpec((D, D), lambda i: (0, 0)),
            pl.BlockSpec((D, D), lambda i: (0, 0)),
            pl.BlockSpec((1, D), lambda i: (0, 0)),
        ],
        out_specs=pl.BlockSpec((BR, D), lambda i: (i, 0)),
        out_shape=jax.ShapeDtypeStruct((N, D), jnp.float32),
    )(h, agg, deg, wst, wnt, b)


def _split_pad(h):
    hp = jnp.pad(h, ((0, NPAD - N), (0, 0)))
    return hp[:, :LHALF], hp[:, LHALF:]


def _recombine(a0, a1):
    return jnp.concatenate([a0[:N], a1[:N]], axis=1)


def kernel(inputs, edge_index, W_self0, W_neigh0, b0, W_self1, W_neigh1, b1):
    x = inputs
    src = jnp.concatenate(
        [edge_index[0], jnp.full((EPAD - E,), N, dtype=jnp.int32)])
    dst = jnp.concatenate(
        [edge_index[1], jnp.full((EPAD - E,), N, dtype=jnp.int32)])

    zrow = jnp.zeros((STRIPE, LHALF), jnp.float32)
    zdeg = jnp.zeros((NPAD,), jnp.float32)

    agg_first = _make_sc_aggregate(compute_deg=True)
    agg_rest = _make_sc_aggregate(compute_deg=False)

    x0, x1 = _split_pad(x)
    a0, a1, degv = agg_first(x0, x1, src, dst, zrow, zdeg)
    agg0 = _recombine(a0, a1)
    deg = degv[:N].reshape(N, 1)

    h1 = _tc_layer(x, agg0, deg, W_self0.T, W_neigh0.T, b0.reshape(1, D))

    h10, h11 = _split_pad(h1)
    a0, a1 = agg_rest(h10, h11, src, dst, zrow, zdeg)
    agg1 = _recombine(a0, a1)

    out = _tc_layer(h1, agg1, deg, W_self1.T, W_neigh1.T, b1.reshape(1, D))
    return out


# speedup vs baseline: 3.2014x; 1.3214x over previous
import jax
import jax.numpy as jnp
from jax import lax
from jax.experimental import pallas as pl
from jax.experimental.pallas import tpu as pltpu
from jax.experimental.pallas import tpu_sc as plsc

N = 10000
D = 256
E = 160000

LHALF = 128
NPAD = 10240
EPAD = 163840
KCH = 128
IBLK = 16
NSUB = 16
EDGES_PER_TILE = EPAD // NSUB
CHUNKS = EDGES_PER_TILE // KCH
STRIPE = NPAD // NSUB


def _make_sc_aggregate(compute_deg: bool):
    out_type = [jax.ShapeDtypeStruct((NPAD, LHALF), jnp.float32),
                jax.ShapeDtypeStruct((NPAD, LHALF), jnp.float32)]
    if compute_deg:
        out_type.append(jax.ShapeDtypeStruct((NPAD,), jnp.float32))
        out_type.append(jax.ShapeDtypeStruct((NSUB * NPAD,), jnp.float32))

    scratch = [
        pltpu.VMEM((IBLK, KCH), jnp.int32),
        pltpu.VMEM((IBLK, KCH), jnp.int32),
        pltpu.VMEM((KCH, LHALF), jnp.float32),
        pltpu.VMEM((KCH, LHALF), jnp.float32),
        pltpu.VMEM_SHARED((NPAD, LHALF), jnp.float32),
        pltpu.SemaphoreType.DMA,
        pltpu.SemaphoreType.DMA,
    ]
    if compute_deg:
        scratch += [
            pltpu.VMEM((NPAD,), jnp.float32),
            pltpu.VMEM((STRIPE,), jnp.float32),
            pltpu.VMEM((STRIPE,), jnp.float32),
        ]

    def body(x0_hbm, x1_hbm, src_hbm, dst_hbm, zrow_hbm, zdeg_hbm, *rest):
        if compute_deg:
            agg0_out, agg1_out, deg_out, degp_out = rest[:4]
            (src_v, dst_v, rows0_v, rows1_v, agg_sh, sem0, sem1,
             deg_local, acc_v, tmp_v) = rest[4:]
        else:
            agg0_out, agg1_out = rest[0], rest[1]
            (src_v, dst_v, rows0_v, rows1_v, agg_sh,
             sem0, sem1) = rest[2:]

        c = lax.axis_index("c")
        s = lax.axis_index("s")
        stripe = pl.ds(s * STRIPE, STRIPE)

        pltpu.sync_copy(zrow_hbm, agg_sh.at[stripe])
        if compute_deg:
            @pl.when(c == 0)
            def _():
                pltpu.sync_copy(zdeg_hbm, deg_local)
        plsc.subcore_barrier()

        ones16 = jnp.full((16,), 1.0, dtype=jnp.float32)
        rows = [rows0_v, rows1_v]
        sems = [sem0, sem1]

        def gather(i, b):
            @pl.when(c == 0)
            def _():
                pltpu.async_copy(x0_hbm.at[src_v.at[i]], rows[b], sems[b])

            @pl.when(c == 1)
            def _():
                pltpu.async_copy(x1_hbm.at[src_v.at[i]], rows[b], sems[b])

        def consume(i, b):
            pltpu.make_async_copy(x0_hbm.at[src_v.at[i]], rows[b],
                                  sems[b]).wait()
            pltpu.sync_copy(rows[b], agg_sh.at[dst_v.at[i]], add=True)
            if compute_deg:
                @pl.when(c == 0)
                def _():
                    for j in range(KCH // 16):
                        idx16 = dst_v[i, pl.ds(j * 16, 16)]
                        plsc.addupdate_scatter(deg_local, [idx16], ones16)

        def block(blk, carry):
            pltpu.sync_copy(src_hbm.at[s, pl.ds(blk * IBLK, IBLK)], src_v)
            pltpu.sync_copy(dst_hbm.at[s, pl.ds(blk * IBLK, IBLK)], dst_v)
            gather(0, 0)

            def chunk2(i2, carry2):
                for b in range(2):
                    i = i2 * 2 + b

                    @pl.when(i + 1 < IBLK)
                    def _():
                        gather(i + 1, 1 - b)

                    consume(i, b)
                return carry2

            lax.fori_loop(0, IBLK // 2, chunk2, 0)
            return carry

        lax.fori_loop(0, CHUNKS // IBLK, block, 0)

        if compute_deg:
            @pl.when(c == 0)
            def _():
                pltpu.sync_copy(deg_local,
                                degp_out.at[pl.ds(s * NPAD, NPAD)])
        plsc.subcore_barrier()

        @pl.when(c == 0)
        def _():
            pltpu.sync_copy(agg_sh.at[stripe], agg0_out.at[stripe])

        @pl.when(c == 1)
        def _():
            pltpu.sync_copy(agg_sh.at[stripe], agg1_out.at[stripe])

        if compute_deg:
            @pl.when(c == 0)
            def _():
                pltpu.sync_copy(degp_out.at[pl.ds(s * STRIPE, STRIPE)],
                                acc_v)
                for t in range(1, NSUB):
                    pltpu.sync_copy(
                        degp_out.at[pl.ds(t * NPAD + s * STRIPE, STRIPE)],
                        tmp_v)

                    def red(j, carry):
                        sl = pl.ds(j * 16, 16)
                        acc_v[sl] = acc_v[sl] + tmp_v[sl]
                        return carry

                    lax.fori_loop(0, STRIPE // 16, red, 0)
                pltpu.sync_copy(acc_v, deg_out.at[stripe])

    return pl.kernel(
        body,
        mesh=plsc.VectorSubcoreMesh(core_axis_name="c", subcore_axis_name="s"),
        out_type=out_type,
        scratch_types=scratch,
        compiler_params=pltpu.CompilerParams(needs_layout_passes=False),
    )


def _tc_layer_body(h_ref, agg_ref, deg_ref, wst_ref, wnt_ref, b_ref, out_ref):
    deg = jnp.maximum(deg_ref[...], 1.0)
    hn = agg_ref[...] / deg
    acc = jnp.dot(h_ref[...], wst_ref[...], preferred_element_type=jnp.float32)
    acc = acc + jnp.dot(hn, wnt_ref[...], preferred_element_type=jnp.float32)
    out_ref[...] = jnp.maximum(acc + b_ref[...], 0.0)


def _tc_layer(h, agg, deg, wst, wnt, b):
    BR = 1000
    grid = (N // BR,)
    return pl.pallas_call(
        _tc_layer_body,
        grid=grid,
        in_specs=[
            pl.BlockSpec((BR, D), lambda i: (i, 0)),
            pl.BlockSpec((BR, D), lambda i: (i, 0)),
            pl.BlockSpec((BR, 1), lambda i: (i, 0)),
            pl.BlockSpec((D, D), lambda i: (0, 0)),
            pl.BlockSpec((D, D), lambda i: (0, 0)),
            pl.BlockSpec((1, D), lambda i: (0, 0)),
        ],
        out_specs=pl.BlockSpec((BR, D), lambda i: (i, 0)),
        out_shape=jax.ShapeDtypeStruct((N, D), jnp.float32),
    )(h, agg, deg, wst, wnt, b)


def _split_pad(h):
    hp = jnp.pad(h, ((0, NPAD - N), (0, 0)))
    return hp[:, :LHALF], hp[:, LHALF:]


def _recombine(a0, a1):
    return jnp.concatenate([a0[:N], a1[:N]], axis=1)


def kernel(inputs, edge_index, W_self0, W_neigh0, b0, W_self1, W_neigh1, b1):
    x = inputs
    src = jnp.concatenate(
        [edge_index[0], jnp.full((EPAD - E,), N, dtype=jnp.int32)]
    ).reshape(NSUB, CHUNKS, KCH)
    dst = jnp.concatenate(
        [edge_index[1], jnp.full((EPAD - E,), N, dtype=jnp.int32)]
    ).reshape(NSUB, CHUNKS, KCH)

    zrow = jnp.zeros((STRIPE, LHALF), jnp.float32)
    zdeg = jnp.zeros((NPAD,), jnp.float32)

    agg_first = _make_sc_aggregate(compute_deg=True)
    agg_rest = _make_sc_aggregate(compute_deg=False)

    x0, x1 = _split_pad(x)
    a0, a1, degv, _ = agg_first(x0, x1, src, dst, zrow, zdeg)
    agg0 = _recombine(a0, a1)
    deg = degv[:N].reshape(N, 1)

    h1 = _tc_layer(x, agg0, deg, W_self0.T, W_neigh0.T, b0.reshape(1, D))

    h10, h11 = _split_pad(h1)
    a0, a1 = agg_rest(h10, h11, src, dst, zrow, zdeg)
    agg1 = _recombine(a0, a1)

    out = _tc_layer(h1, agg1, deg, W_self1.T, W_neigh1.T, b1.reshape(1, D))
    return out


# speedup vs baseline: 3.2021x; 1.0002x over previous
import jax
import jax.numpy as jnp
from jax import lax
from jax.experimental import pallas as pl
from jax.experimental.pallas import tpu as pltpu
from jax.experimental.pallas import tpu_sc as plsc

N = 10000
D = 256
E = 160000

LHALF = 128
NPAD = 10240
EPAD = 163840
KCH = 128
IBLK = 16
NSUB = 16
EDGES_PER_TILE = EPAD // NSUB
CHUNKS = EDGES_PER_TILE // KCH
STRIPE = NPAD // NSUB


def _make_sc_aggregate(compute_deg: bool):
    out_type = [jax.ShapeDtypeStruct((NPAD, LHALF), jnp.float32),
                jax.ShapeDtypeStruct((NPAD, LHALF), jnp.float32)]
    if compute_deg:
        out_type.append(jax.ShapeDtypeStruct((NPAD,), jnp.float32))
        out_type.append(jax.ShapeDtypeStruct((NSUB * NPAD,), jnp.float32))

    scratch = [
        pltpu.VMEM((IBLK, KCH), jnp.int32),
        pltpu.VMEM((IBLK, KCH), jnp.int32),
        pltpu.VMEM((KCH, LHALF), jnp.float32),
        pltpu.VMEM((KCH, LHALF), jnp.float32),
        pltpu.VMEM_SHARED((NPAD, LHALF), jnp.float32),
        pltpu.SemaphoreType.DMA,
        pltpu.SemaphoreType.DMA,
        pltpu.SemaphoreType.DMA,
        pltpu.SemaphoreType.DMA,
    ]
    if compute_deg:
        scratch += [
            pltpu.VMEM((NPAD,), jnp.float32),
            pltpu.VMEM((STRIPE,), jnp.float32),
            pltpu.VMEM((STRIPE,), jnp.float32),
        ]

    def body(x0_hbm, x1_hbm, src_hbm, dst_hbm, zrow_hbm, zdeg_hbm, *rest):
        if compute_deg:
            agg0_out, agg1_out, deg_out, degp_out = rest[:4]
            (src_v, dst_v, rows0_v, rows1_v, agg_sh, gsem0, gsem1,
             ssem0, ssem1, deg_local, acc_v, tmp_v) = rest[4:]
        else:
            agg0_out, agg1_out = rest[0], rest[1]
            (src_v, dst_v, rows0_v, rows1_v, agg_sh,
             gsem0, gsem1, ssem0, ssem1) = rest[2:]

        c = lax.axis_index("c")
        s = lax.axis_index("s")
        stripe = pl.ds(s * STRIPE, STRIPE)

        pltpu.sync_copy(zrow_hbm, agg_sh.at[stripe])
        if compute_deg:
            @pl.when(c == 0)
            def _():
                pltpu.sync_copy(zdeg_hbm, deg_local)
        plsc.subcore_barrier()

        ones16 = jnp.full((16,), 1.0, dtype=jnp.float32)
        rows = [rows0_v, rows1_v]
        gsems = [gsem0, gsem1]
        ssems = [ssem0, ssem1]

        def gather(i, b):
            @pl.when(c == 0)
            def _():
                pltpu.async_copy(x0_hbm.at[src_v.at[i]], rows[b], gsems[b])

            @pl.when(c == 1)
            def _():
                pltpu.async_copy(x1_hbm.at[src_v.at[i]], rows[b], gsems[b])

        def drain_scatter(b):
            pltpu.make_async_copy(rows[b], agg_sh.at[dst_v.at[0]],
                                  ssems[b]).wait()

        def consume(i, b):
            pltpu.make_async_copy(x0_hbm.at[src_v.at[i]], rows[b],
                                  gsems[b]).wait()
            pltpu.async_copy(rows[b], agg_sh.at[dst_v.at[i]], ssems[b],
                             add=True)
            if compute_deg:
                @pl.when(c == 0)
                def _():
                    for j in range(KCH // 16):
                        idx16 = dst_v[i, pl.ds(j * 16, 16)]
                        plsc.addupdate_scatter(deg_local, [idx16], ones16)

        def block(blk, carry):
            pltpu.sync_copy(src_hbm.at[s, pl.ds(blk * IBLK, IBLK)], src_v)
            pltpu.sync_copy(dst_hbm.at[s, pl.ds(blk * IBLK, IBLK)], dst_v)
            gather(0, 0)

            def chunk2(i2, carry2):
                for b in range(2):
                    i = i2 * 2 + b

                    if b == 0:
                        @pl.when(i2 >= 1)
                        def _():
                            drain_scatter(1)
                    else:
                        drain_scatter(0)

                    @pl.when(i + 1 < IBLK)
                    def _():
                        gather(i + 1, 1 - b)

                    consume(i, b)
                return carry2

            lax.fori_loop(0, IBLK // 2, chunk2, 0)
            drain_scatter(1)
            return carry

        lax.fori_loop(0, CHUNKS // IBLK, block, 0)

        if compute_deg:
            @pl.when(c == 0)
            def _():
                pltpu.sync_copy(deg_local,
                                degp_out.at[pl.ds(s * NPAD, NPAD)])
        plsc.subcore_barrier()

        @pl.when(c == 0)
        def _():
            pltpu.sync_copy(agg_sh.at[stripe], agg0_out.at[stripe])

        @pl.when(c == 1)
        def _():
            pltpu.sync_copy(agg_sh.at[stripe], agg1_out.at[stripe])

        if compute_deg:
            @pl.when(c == 0)
            def _():
                pltpu.sync_copy(degp_out.at[pl.ds(s * STRIPE, STRIPE)],
                                acc_v)
                for t in range(1, NSUB):
                    pltpu.sync_copy(
                        degp_out.at[pl.ds(t * NPAD + s * STRIPE, STRIPE)],
                        tmp_v)

                    def red(j, carry):
                        sl = pl.ds(j * 16, 16)
                        acc_v[sl] = acc_v[sl] + tmp_v[sl]
                        return carry

                    lax.fori_loop(0, STRIPE // 16, red, 0)
                pltpu.sync_copy(acc_v, deg_out.at[stripe])

    return pl.kernel(
        body,
        mesh=plsc.VectorSubcoreMesh(core_axis_name="c", subcore_axis_name="s"),
        out_type=out_type,
        scratch_types=scratch,
        compiler_params=pltpu.CompilerParams(needs_layout_passes=False),
    )


def _tc_layer_body(h_ref, agg_ref, deg_ref, wst_ref, wnt_ref, b_ref, out_ref):
    deg = jnp.maximum(deg_ref[...], 1.0)
    hn = agg_ref[...] / deg
    acc = jnp.dot(h_ref[...], wst_ref[...], preferred_element_type=jnp.float32)
    acc = acc + jnp.dot(hn, wnt_ref[...], preferred_element_type=jnp.float32)
    out_ref[...] = jnp.maximum(acc + b_ref[...], 0.0)


def _tc_layer(h, agg, deg, wst, wnt, b):
    BR = 1000
    grid = (N // BR,)
    return pl.pallas_call(
        _tc_layer_body,
        grid=grid,
        in_specs=[
            pl.BlockSpec((BR, D), lambda i: (i, 0)),
            pl.BlockSpec((BR, D), lambda i: (i, 0)),
            pl.BlockSpec((BR, 1), lambda i: (i, 0)),
            pl.BlockSpec((D, D), lambda i: (0, 0)),
            pl.BlockSpec((D, D), lambda i: (0, 0)),
            pl.BlockSpec((1, D), lambda i: (0, 0)),
        ],
        out_specs=pl.BlockSpec((BR, D), lambda i: (i, 0)),
        out_shape=jax.ShapeDtypeStruct((N, D), jnp.float32),
    )(h, agg, deg, wst, wnt, b)


def _split_pad(h):
    hp = jnp.pad(h, ((0, NPAD - N), (0, 0)))
    return hp[:, :LHALF], hp[:, LHALF:]


def _recombine(a0, a1):
    return jnp.concatenate([a0[:N], a1[:N]], axis=1)


def kernel(inputs, edge_index, W_self0, W_neigh0, b0, W_self1, W_neigh1, b1):
    x = inputs
    src = jnp.concatenate(
        [edge_index[0], jnp.full((EPAD - E,), N, dtype=jnp.int32)]
    ).reshape(NSUB, CHUNKS, KCH)
    dst = jnp.concatenate(
        [edge_index[1], jnp.full((EPAD - E,), N, dtype=jnp.int32)]
    ).reshape(NSUB, CHUNKS, KCH)

    zrow = jnp.zeros((STRIPE, LHALF), jnp.float32)
    zdeg = jnp.zeros((NPAD,), jnp.float32)

    agg_first = _make_sc_aggregate(compute_deg=True)
    agg_rest = _make_sc_aggregate(compute_deg=False)

    x0, x1 = _split_pad(x)
    a0, a1, degv, _ = agg_first(x0, x1, src, dst, zrow, zdeg)
    agg0 = _recombine(a0, a1)
    deg = degv[:N].reshape(N, 1)

    h1 = _tc_layer(x, agg0, deg, W_self0.T, W_neigh0.T, b0.reshape(1, D))

    h10, h11 = _split_pad(h1)
    a0, a1 = agg_rest(h10, h11, src, dst, zrow, zdeg)
    agg1 = _recombine(a0, a1)

    out = _tc_layer(h1, agg1, deg, W_self1.T, W_neigh1.T, b1.reshape(1, D))
    return out


# speedup vs baseline: 3.4973x; 1.0922x over previous
import functools

import jax
import jax.numpy as jnp
from jax import lax
from jax.experimental import pallas as pl
from jax.experimental.pallas import tpu as pltpu
from jax.experimental.pallas import tpu_sc as plsc

N = 10000
D = 256
E = 160000

LHALF = 128
NPAD = 10240
EPAD = 163840
KCH = 128
IBLK = 16
NSUB = 16
EDGES_PER_TILE = EPAD // NSUB
CHUNKS = EDGES_PER_TILE // KCH
STRIPE = NPAD // NSUB


def _make_sc_aggregate(compute_deg: bool):
    out_type = [jax.ShapeDtypeStruct((NPAD, LHALF), jnp.float32),
                jax.ShapeDtypeStruct((NPAD, LHALF), jnp.float32)]
    if compute_deg:
        out_type.append(jax.ShapeDtypeStruct((NPAD,), jnp.float32))
        out_type.append(jax.ShapeDtypeStruct((NSUB * NPAD,), jnp.float32))

    scratch = [
        pltpu.VMEM((IBLK, KCH), jnp.int32),
        pltpu.VMEM((IBLK, KCH), jnp.int32),
        pltpu.VMEM((KCH, LHALF), jnp.float32),
        pltpu.VMEM((KCH, LHALF), jnp.float32),
        pltpu.VMEM_SHARED((NPAD, LHALF), jnp.float32),
        pltpu.SemaphoreType.DMA,
        pltpu.SemaphoreType.DMA,
        pltpu.SemaphoreType.DMA,
        pltpu.SemaphoreType.DMA,
    ]
    if compute_deg:
        scratch += [
            pltpu.VMEM((NPAD,), jnp.float32),
            pltpu.VMEM((STRIPE,), jnp.float32),
            pltpu.VMEM((STRIPE,), jnp.float32),
        ]

    def body(x0_hbm, x1_hbm, src_hbm, dst_hbm, zrow_hbm, zdeg_hbm, *rest):
        if compute_deg:
            agg0_out, agg1_out, deg_out, degp_out = rest[:4]
            (src_v, dst_v, rows0_v, rows1_v, agg_sh, gsem0, gsem1,
             ssem0, ssem1, deg_local, acc_v, tmp_v) = rest[4:]
        else:
            agg0_out, agg1_out = rest[0], rest[1]
            (src_v, dst_v, rows0_v, rows1_v, agg_sh,
             gsem0, gsem1, ssem0, ssem1) = rest[2:]

        c = lax.axis_index("c")
        s = lax.axis_index("s")
        stripe = pl.ds(s * STRIPE, STRIPE)

        pltpu.sync_copy(zrow_hbm, agg_sh.at[stripe])
        if compute_deg:
            @pl.when(c == 0)
            def _():
                pltpu.sync_copy(zdeg_hbm, deg_local)
        plsc.subcore_barrier()

        ones16 = jnp.full((16,), 1.0, dtype=jnp.float32)
        rows = [rows0_v, rows1_v]
        gsems = [gsem0, gsem1]
        ssems = [ssem0, ssem1]

        def gather(i, b):
            @pl.when(c == 0)
            def _():
                pltpu.async_copy(x0_hbm.at[src_v.at[i]], rows[b], gsems[b])

            @pl.when(c == 1)
            def _():
                pltpu.async_copy(x1_hbm.at[src_v.at[i]], rows[b], gsems[b])

        def drain_scatter(b):
            pltpu.make_async_copy(rows[b], agg_sh.at[dst_v.at[0]],
                                  ssems[b]).wait()

        def consume(i, b):
            pltpu.make_async_copy(x0_hbm.at[src_v.at[i]], rows[b],
                                  gsems[b]).wait()
            pltpu.async_copy(rows[b], agg_sh.at[dst_v.at[i]], ssems[b],
                             add=True)
            if compute_deg:
                @pl.when(c == 0)
                def _():
                    for j in range(KCH // 16):
                        idx16 = dst_v[i, pl.ds(j * 16, 16)]
                        plsc.addupdate_scatter(deg_local, [idx16], ones16)

        def block(blk, carry):
            pltpu.sync_copy(src_hbm.at[s, pl.ds(blk * IBLK, IBLK)], src_v)
            pltpu.sync_copy(dst_hbm.at[s, pl.ds(blk * IBLK, IBLK)], dst_v)
            gather(0, 0)

            def chunk2(i2, carry2):
                for b in range(2):
                    i = i2 * 2 + b

                    if b == 0:
                        @pl.when(i2 >= 1)
                        def _():
                            drain_scatter(1)
                    else:
                        drain_scatter(0)

                    @pl.when(i + 1 < IBLK)
                    def _():
                        gather(i + 1, 1 - b)

                    consume(i, b)
                return carry2

            lax.fori_loop(0, IBLK // 2, chunk2, 0)
            drain_scatter(1)
            return carry

        lax.fori_loop(0, CHUNKS // IBLK, block, 0)

        if compute_deg:
            @pl.when(c == 0)
            def _():
                pltpu.sync_copy(deg_local,
                                degp_out.at[pl.ds(s * NPAD, NPAD)])
        plsc.subcore_barrier()

        @pl.when(c == 0)
        def _():
            pltpu.sync_copy(agg_sh.at[stripe], agg0_out.at[stripe])

        @pl.when(c == 1)
        def _():
            pltpu.sync_copy(agg_sh.at[stripe], agg1_out.at[stripe])

        if compute_deg:
            @pl.when(c == 0)
            def _():
                pltpu.sync_copy(degp_out.at[pl.ds(s * STRIPE, STRIPE)],
                                acc_v)
                for t in range(1, NSUB):
                    pltpu.sync_copy(
                        degp_out.at[pl.ds(t * NPAD + s * STRIPE, STRIPE)],
                        tmp_v)

                    def red(j, carry):
                        sl = pl.ds(j * 16, 16)
                        acc_v[sl] = acc_v[sl] + tmp_v[sl]
                        return carry

                    lax.fori_loop(0, STRIPE // 16, red, 0)
                pltpu.sync_copy(acc_v, deg_out.at[stripe])

    return pl.kernel(
        body,
        mesh=plsc.VectorSubcoreMesh(core_axis_name="c", subcore_axis_name="s"),
        out_type=out_type,
        scratch_types=scratch,
        compiler_params=pltpu.CompilerParams(needs_layout_passes=False),
    )


BR = 1000


def _tc_layer_body(split_out, h0_ref, h1_ref, a0_ref, a1_ref, deg_ref,
                   wst_ref, wnt_ref, b_ref, *out_refs):
    h = jnp.concatenate([h0_ref[...], h1_ref[...]], axis=1)
    agg = jnp.concatenate([a0_ref[...], a1_ref[...]], axis=1)
    deg = jnp.maximum(deg_ref[...], 1.0)
    hn = agg / deg
    acc = jnp.dot(h, wst_ref[...], preferred_element_type=jnp.float32)
    acc = acc + jnp.dot(hn, wnt_ref[...], preferred_element_type=jnp.float32)
    res = jnp.maximum(acc + b_ref[...], 0.0)
    if split_out:
        out_refs[0][...] = res[:, :LHALF]
        out_refs[1][...] = res[:, LHALF:]
    else:
        out_refs[0][...] = res


def _tc_layer(h0, h1, a0, a1, deg, wst, wnt, b, split_out):
    if split_out:
        out_shape = [jax.ShapeDtypeStruct((NPAD, LHALF), jnp.float32),
                     jax.ShapeDtypeStruct((NPAD, LHALF), jnp.float32)]
        out_specs = [pl.BlockSpec((BR, LHALF), lambda i: (i, 0)),
                     pl.BlockSpec((BR, LHALF), lambda i: (i, 0))]
    else:
        out_shape = jax.ShapeDtypeStruct((N, D), jnp.float32)
        out_specs = pl.BlockSpec((BR, D), lambda i: (i, 0))
    half = pl.BlockSpec((BR, LHALF), lambda i: (i, 0))
    return pl.pallas_call(
        functools.partial(_tc_layer_body, split_out),
        grid=(N // BR,),
        in_specs=[
            half, half, half, half,
            pl.BlockSpec((BR, 1), lambda i: (i, 0)),
            pl.BlockSpec((D, D), lambda i: (0, 0)),
            pl.BlockSpec((D, D), lambda i: (0, 0)),
            pl.BlockSpec((1, D), lambda i: (0, 0)),
        ],
        out_specs=out_specs,
        out_shape=out_shape,
    )(h0, h1, a0, a1, deg, wst, wnt, b)


def _split_pad(h):
    hp = jnp.pad(h, ((0, NPAD - N), (0, 0)))
    return hp[:, :LHALF], hp[:, LHALF:]


def kernel(inputs, edge_index, W_self0, W_neigh0, b0, W_self1, W_neigh1, b1):
    x = inputs
    src = jnp.concatenate(
        [edge_index[0], jnp.full((EPAD - E,), N, dtype=jnp.int32)]
    ).reshape(NSUB, CHUNKS, KCH)
    dst = jnp.concatenate(
        [edge_index[1], jnp.full((EPAD - E,), N, dtype=jnp.int32)]
    ).reshape(NSUB, CHUNKS, KCH)

    zrow = jnp.zeros((STRIPE, LHALF), jnp.float32)
    zdeg = jnp.zeros((NPAD,), jnp.float32)

    agg_first = _make_sc_aggregate(compute_deg=True)
    agg_rest = _make_sc_aggregate(compute_deg=False)

    x0, x1 = _split_pad(x)
    a0, a1, degv, _ = agg_first(x0, x1, src, dst, zrow, zdeg)
    deg = degv[:N].reshape(N, 1)

    h10, h11 = _tc_layer(x0, x1, a0, a1, deg, W_self0.T, W_neigh0.T,
                         b0.reshape(1, D), split_out=True)

    a0, a1 = agg_rest(h10, h11, src, dst, zrow, zdeg)

    out = _tc_layer(h10, h11, a0, a1, deg, W_self1.T, W_neigh1.T,
                    b1.reshape(1, D), split_out=False)
    return out
